# baseline (speedup 1.0000x reference)
import jax
import jax.numpy as jnp
from jax.experimental import pallas as pl

_NUM_EMB = 1024
_EMB_DIM = 64
_COMMIT = 0.25


def _vq_kernel(z_ref, cb_ref, zq_ref, idx_ref, loss_ref):
    z_t = z_ref[0].reshape(_EMB_DIM, 1024)
    flat = z_t.T
    cb = cb_ref[...]

    f_sq = jnp.sum(flat * flat, axis=1, keepdims=True)
    c_sq = jnp.sum(cb * cb, axis=1)[None, :]
    dot = jax.lax.dot_general(
        flat, cb, (((1,), (1,)), ((), ())),
        preferred_element_type=jnp.float32)
    sq = (f_sq + c_sq) - 2.0 * dot
    dists = jnp.sqrt(jnp.maximum(sq, 0.0))

    minval = jnp.min(dists, axis=1, keepdims=True)
    code_iota = jax.lax.broadcasted_iota(jnp.int32, (1024, _NUM_EMB), 1)
    idx = jnp.min(
        jnp.where(dists == minval, code_iota, _NUM_EMB), axis=1)
    idx_ref[...] = idx.reshape(1, 8, 128)

    onehot_t = (jax.lax.broadcasted_iota(jnp.int32, (_NUM_EMB, 1024), 0)
                == idx[None, :]).astype(jnp.float32)
    zq_t = jax.lax.dot_general(
        cb, onehot_t, (((0,), (0,)), ((), ())),
        preferred_element_type=jnp.float32,
        precision=jax.lax.Precision.HIGHEST)
    zq_ref[...] = zq_t.reshape(1, _EMB_DIM, 32, 32)

    diff = zq_t - z_t
    partial = jnp.sum(diff * diff).reshape(1, 1)

    @pl.when(pl.program_id(0) == 0)
    def _init():
        loss_ref[...] = jnp.zeros((1, 1), jnp.float32)

    loss_ref[...] += partial


def kernel(z_e, codebook):
    b, c, h, w = z_e.shape
    n_pos = b * h * w

    zq, idx3, loss_sum = pl.pallas_call(
        _vq_kernel,
        grid=(b,),
        in_specs=[
            pl.BlockSpec((1, c, h, w), lambda i: (i, 0, 0, 0)),
            pl.BlockSpec((_NUM_EMB, _EMB_DIM), lambda i: (0, 0)),
        ],
        out_specs=[
            pl.BlockSpec((1, c, h, w), lambda i: (i, 0, 0, 0)),
            pl.BlockSpec((1, 8, 128), lambda i: (i, 0, 0)),
            pl.BlockSpec((1, 1), lambda i: (0, 0)),
        ],
        out_shape=[
            jax.ShapeDtypeStruct((b, c, h, w), jnp.float32),
            jax.ShapeDtypeStruct((b, 8, 128), jnp.int32),
            jax.ShapeDtypeStruct((1, 1), jnp.float32),
        ],
    )(z_e, codebook)

    loss = loss_sum[0, 0] * ((1.0 + _COMMIT) / (n_pos * _EMB_DIM))
    indices = idx3.reshape(n_pos)
    return (zq, loss, indices)


# speedup vs baseline: 1.1957x; 1.1957x over previous
import jax
import jax.numpy as jnp
from jax.experimental import pallas as pl

_NUM_EMB = 1024
_EMB_DIM = 64
_COMMIT = 0.25


def _vq_kernel(z_ref, cb_ref, iota_col_ref, zq_ref, idx_ref, loss_ref):
    z_t = z_ref[0]
    cb = cb_ref[...]
    iota_col = iota_col_ref[...]
    flat = z_t.T

    f_sq = jnp.sum(flat * flat, axis=1, keepdims=True)
    c_sq = jnp.sum(cb * cb, axis=1)[None, :]
    dot2 = jax.lax.dot_general(
        flat, cb + cb, (((1,), (1,)), ((), ())),
        preferred_element_type=jnp.float32)
    sq = (f_sq + c_sq) - dot2
    dists = jnp.sqrt(jnp.maximum(sq, 0.0))

    minval = jnp.min(dists, axis=1, keepdims=True)
    idx = jnp.min(
        jnp.where(dists == minval, iota_col.T, _NUM_EMB), axis=1)
    idx_ref[...] = idx.reshape(1, 8, 128)

    onehot_t = jnp.where(iota_col == idx[None, :], 1.0, 0.0)
    zq_t = jax.lax.dot_general(
        cb, onehot_t, (((0,), (0,)), ((), ())),
        preferred_element_type=jnp.float32,
        precision=jax.lax.Precision.HIGHEST)
    zq_ref[...] = zq_t[None]

    diff = zq_t - z_t
    partial = jnp.sum(diff * diff).reshape(1, 1)

    @pl.when(pl.program_id(0) == 0)
    def _init():
        loss_ref[...] = jnp.zeros((1, 1), jnp.float32)

    loss_ref[...] += partial


def kernel(z_e, codebook):
    b, c, h, w = z_e.shape
    n_pos = b * h * w
    hw = h * w
    z3 = z_e.reshape(b, c, hw)
    iota_col = jax.lax.broadcasted_iota(jnp.int32, (_NUM_EMB, 1), 0)

    zq3, idx3, loss_sum = pl.pallas_call(
        _vq_kernel,
        grid=(b,),
        in_specs=[
            pl.BlockSpec((1, c, hw), lambda i: (i, 0, 0)),
            pl.BlockSpec((_NUM_EMB, _EMB_DIM), lambda i: (0, 0)),
            pl.BlockSpec((_NUM_EMB, 1), lambda i: (0, 0)),
        ],
        out_specs=[
            pl.BlockSpec((1, c, hw), lambda i: (i, 0, 0)),
            pl.BlockSpec((1, 8, 128), lambda i: (i, 0, 0)),
            pl.BlockSpec((1, 1), lambda i: (0, 0)),
        ],
        out_shape=[
            jax.ShapeDtypeStruct((b, c, hw), jnp.float32),
            jax.ShapeDtypeStruct((b, 8, 128), jnp.int32),
            jax.ShapeDtypeStruct((1, 1), jnp.float32),
        ],
    )(z3, codebook, iota_col)

    loss = loss_sum[0, 0] * ((1.0 + _COMMIT) / (n_pos * _EMB_DIM))
    indices = idx3.reshape(n_pos)
    zq = zq3.reshape(b, c, h, w)
    return (zq, loss, indices)


# speedup vs baseline: 1.2693x; 1.0615x over previous
import jax
import jax.numpy as jnp
from jax.experimental import pallas as pl

_NUM_EMB = 1024
_EMB_DIM = 64
_COMMIT = 0.25
_BB = 4


def _vq_kernel(z_ref, cb_ref, iota_col_ref, zq_ref, idx_ref, loss_ref):
    cb = cb_ref[...]
    iota_col = iota_col_ref[...]
    c_sq = jnp.sum(cb * cb, axis=1)[None, :]
    cb2 = cb + cb

    partial = jnp.zeros((1, 1), jnp.float32)
    for j in range(_BB):
        z_t = z_ref[j]
        flat = z_t.T

        f_sq = jnp.sum(flat * flat, axis=1, keepdims=True)
        dot2 = jax.lax.dot_general(
            flat, cb2, (((1,), (1,)), ((), ())),
            preferred_element_type=jnp.float32)
        sq = (f_sq + c_sq) - dot2
        dists = jnp.sqrt(jnp.maximum(sq, 0.0))

        minval = jnp.min(dists, axis=1, keepdims=True)
        idx = jnp.min(
            jnp.where(dists == minval, iota_col.T, _NUM_EMB), axis=1)
        idx_ref[j] = idx.reshape(8, 128)

        onehot_t = jnp.where(iota_col == idx[None, :], 1.0, 0.0)
        zq_t = jax.lax.dot_general(
            cb, onehot_t, (((0,), (0,)), ((), ())),
            preferred_element_type=jnp.float32,
            precision=jax.lax.Precision.HIGHEST)
        zq_ref[j] = zq_t

        diff = zq_t - z_t
        partial += jnp.sum(diff * diff).reshape(1, 1)

    @pl.when(pl.program_id(0) == 0)
    def _init():
        loss_ref[...] = jnp.zeros((1, 1), jnp.float32)

    loss_ref[...] += partial


def kernel(z_e, codebook):
    b, c, h, w = z_e.shape
    n_pos = b * h * w
    hw = h * w
    z3 = z_e.reshape(b, c, hw)
    iota_col = jax.lax.broadcasted_iota(jnp.int32, (_NUM_EMB, 1), 0)

    zq3, idx3, loss_sum = pl.pallas_call(
        _vq_kernel,
        grid=(b // _BB,),
        in_specs=[
            pl.BlockSpec((_BB, c, hw), lambda i: (i, 0, 0)),
            pl.BlockSpec((_NUM_EMB, _EMB_DIM), lambda i: (0, 0)),
            pl.BlockSpec((_NUM_EMB, 1), lambda i: (0, 0)),
        ],
        out_specs=[
            pl.BlockSpec((_BB, c, hw), lambda i: (i, 0, 0)),
            pl.BlockSpec((_BB, 8, 128), lambda i: (i, 0, 0)),
            pl.BlockSpec((1, 1), lambda i: (0, 0)),
        ],
        out_shape=[
            jax.ShapeDtypeStruct((b, c, hw), jnp.float32),
            jax.ShapeDtypeStruct((b, 8, 128), jnp.int32),
            jax.ShapeDtypeStruct((1, 1), jnp.float32),
        ],
    )(z3, codebook, iota_col)

    loss = loss_sum[0, 0] * ((1.0 + _COMMIT) / (n_pos * _EMB_DIM))
    indices = idx3.reshape(n_pos)
    zq = zq3.reshape(b, c, h, w)
    return (zq, loss, indices)


# speedup vs baseline: 1.3703x; 1.0796x over previous
import jax
import jax.numpy as jnp
from jax.experimental import pallas as pl

_NUM_EMB = 1024
_EMB_DIM = 64
_COMMIT = 0.25
_BB = 4


def _vq_kernel(z_ref, cb_ref, iota_col_ref, zq_ref, idx_ref, loss_ref):
    cb = cb_ref[...]
    iota_col = iota_col_ref[...]
    c_sq = jnp.sum(cb * cb, axis=1)[None, :]
    cb2 = cb + cb

    partial = jnp.zeros((1, 1), jnp.float32)
    for j in range(_BB):
        z_t = z_ref[j]
        flat = z_t.T

        f_sq = jnp.sum(flat * flat, axis=1, keepdims=True)
        dot2 = jax.lax.dot_general(
            flat, cb2, (((1,), (1,)), ((), ())),
            preferred_element_type=jnp.float32)
        sq = (f_sq + c_sq) - dot2
        dists = jnp.sqrt(jnp.maximum(sq, 0.0))

        idx = jnp.argmin(dists, axis=1).astype(jnp.int32)
        idx_ref[j] = idx.reshape(8, 128)

        onehot_t = jnp.where(iota_col == idx[None, :], 1.0, 0.0)
        zq_t = jax.lax.dot_general(
            cb, onehot_t, (((0,), (0,)), ((), ())),
            preferred_element_type=jnp.float32,
            precision=jax.lax.Precision.HIGHEST)
        zq_ref[j] = zq_t

        diff = zq_t - z_t
        partial += jnp.sum(diff * diff).reshape(1, 1)

    @pl.when(pl.program_id(0) == 0)
    def _init():
        loss_ref[...] = jnp.zeros((1, 1), jnp.float32)

    loss_ref[...] += partial


def kernel(z_e, codebook):
    b, c, h, w = z_e.shape
    n_pos = b * h * w
    hw = h * w
    z3 = z_e.reshape(b, c, hw)
    iota_col = jax.lax.broadcasted_iota(jnp.int32, (_NUM_EMB, 1), 0)

    zq3, idx3, loss_sum = pl.pallas_call(
        _vq_kernel,
        grid=(b // _BB,),
        in_specs=[
            pl.BlockSpec((_BB, c, hw), lambda i: (i, 0, 0)),
            pl.BlockSpec((_NUM_EMB, _EMB_DIM), lambda i: (0, 0)),
            pl.BlockSpec((_NUM_EMB, 1), lambda i: (0, 0)),
        ],
        out_specs=[
            pl.BlockSpec((_BB, c, hw), lambda i: (i, 0, 0)),
            pl.BlockSpec((_BB, 8, 128), lambda i: (i, 0, 0)),
            pl.BlockSpec((1, 1), lambda i: (0, 0)),
        ],
        out_shape=[
            jax.ShapeDtypeStruct((b, c, hw), jnp.float32),
            jax.ShapeDtypeStruct((b, 8, 128), jnp.int32),
            jax.ShapeDtypeStruct((1, 1), jnp.float32),
        ],
    )(z3, codebook, iota_col)

    loss = loss_sum[0, 0] * ((1.0 + _COMMIT) / (n_pos * _EMB_DIM))
    indices = idx3.reshape(n_pos)
    zq = zq3.reshape(b, c, h, w)
    return (zq, loss, indices)
